# baseline (speedup 1.0000x reference)
import jax
import jax.numpy as jnp
from jax import lax
from jax.experimental import pallas as pl
from jax.experimental.pallas import tpu as pltpu
from jax.experimental.pallas import tpu_sc as plsc

_NC = 2
_NS = 16
_NW = _NC * _NS
_L = 16


def _make_reformat(B, X, Y, Z, C):
    NSHEET_G = B * X
    SPW = NSHEET_G // _NW
    HALF = Y // 2
    LB = 4
    mesh = plsc.VectorSubcoreMesh(core_axis_name="c", subcore_axis_name="s")

    def body(vol2d, table, sx_v, sx1_v, ob0, ob1, sem0, sem1):
        obufs = [ob0, ob1]
        sems = [sem0, sem1]
        cid = lax.axis_index("c")
        sid = lax.axis_index("s")
        wid = cid * _NS + sid
        j = lax.iota(jnp.int32, _L)
        q = j >> 2
        ch = j & 3
        dy = q >> 1
        c_clamp = q & 1
        c_hi = c_clamp + ch * Z
        c_lo = c_hi + dy * (C * Z)

        @pl.loop(0, SPW)
        def sheet_loop(s):
            sg = wid * SPW + s
            x = sg - (sg // X) * X
            sg1 = jnp.where(x == X - 1, sg, sg + 1)
            for h in range(2):
                nl = HALF + 2 if h == 0 else HALF
                y0 = h * HALF
                pltpu.sync_copy(
                    vol2d.at[pl.ds((sg * Y + y0) * C, nl * C)],
                    sx_v.at[pl.ds(0, nl * C)])
                pltpu.sync_copy(
                    vol2d.at[pl.ds((sg1 * Y + y0) * C, nl * C)],
                    sx1_v.at[pl.ds(0, nl * C)])

                @pl.loop(0, HALF // (2 * LB))
                def line_blk(t):
                    for bi in range(2):
                        ob, sem = obufs[bi], sems[bi]
                        yy0 = (2 * t + bi) * LB

                        @pl.when(jnp.logical_or(t > 0, (s * 2 + h) > 0))
                        def _():
                            pltpu.make_async_copy(
                                ob, table.at[pl.ds(0, LB * (Z // 4))], sem
                            ).wait()

                        for li in range(LB):
                            yy = yy0 + li
                            last = jnp.logical_and(yy == HALF - 1, h == 1)
                            cy = jnp.where(last, c_hi, c_lo) + yy * (C * Z)
                            cyz = cy - c_clamp

                            @pl.loop(0, Z, unroll=4)
                            def z_loop(z):
                                idx = jnp.where(z < Z - 1, cy, cyz) + z
                                r = idx >> 7
                                cc = idx & (Z - 1)
                                v0 = plsc.load_gather(sx_v, [r, cc])
                                v1 = plsc.load_gather(sx1_v, [r, cc])
                                orow = li * (Z // 4) + (z >> 2)
                                ob[orow, pl.ds((z & 3) * 32, 16)] = v0
                                ob[orow, pl.ds((z & 3) * 32 + 16, 16)] = v1

                        line_g = sg * Y + y0 + yy0
                        pltpu.async_copy(
                            ob, table.at[pl.ds(line_g * (Z // 4), LB * (Z // 4))],
                            sem)

        for bi in range(2):
            pltpu.make_async_copy(
                obufs[bi], table.at[pl.ds(0, LB * (Z // 4))], sems[bi]).wait()

    return pl.kernel(
        body,
        out_type=jax.ShapeDtypeStruct((B * X * Y * Z // 4, 128), jnp.float32),
        mesh=mesh,
        scratch_types=[
            pltpu.VMEM(((HALF + 2) * C, Z), jnp.float32),
            pltpu.VMEM(((HALF + 2) * C, Z), jnp.float32),
            pltpu.VMEM((LB * (Z // 4), 128), jnp.float32),
            pltpu.VMEM((LB * (Z // 4), 128), jnp.float32),
            pltpu.SemaphoreType.DMA,
            pltpu.SemaphoreType.DMA,
        ],
        compiler_params=pltpu.CompilerParams(needs_layout_passes=False),
    )


def _make_kernel(B, X, Y, Z, C, P, NL):
    PPW = P // _NW
    K = 384
    NCHUNK = PPW // K
    GD = K // 128
    MQ = 8 * C

    mesh = plsc.VectorSubcoreMesh(core_axis_name="c", subcore_axis_name="s")

    def body(table, coords, out,
             coords_v0, coords_v1, idx_v0, idx_v1, vals_v0, vals_v1,
             out_v0, out_v1, sem0, sem1):
        coords_b = [coords_v0, coords_v1]
        idx_b = [idx_v0, idx_v1]
        vals_b = [vals_v0, vals_v1]
        out_b = [out_v0, out_v1]
        sem_b = [sem0, sem1]
        cid = lax.axis_index("c")
        sid = lax.axis_index("s")
        wid = cid * _NS + sid
        batch = (wid * PPW) // (P // B)
        b_off = batch * (X * Y * Z)
        base0 = wid * PPW
        iota = lax.iota(jnp.int32, _L)
        zeros = jnp.zeros((_L,), jnp.float32)
        ones = jnp.ones((_L,), jnp.float32)

        def load_xyz(coords_v, i0):
            x = coords_v[pl.ds(i0, _L)]
            y = coords_v[pl.ds(K + i0, _L)]
            z = coords_v[pl.ds(2 * K + i0, _L)]
            return x, y, z

        def mega_ids(x, y, z):
            xi = x.astype(jnp.int32)
            yi = y.astype(jnp.int32)
            zi = z.astype(jnp.int32)
            x0 = jnp.clip(xi, 0, X - 1)
            y0 = jnp.clip(yi, 0, Y - 1)
            z0 = jnp.clip(zi, 0, Z - 1)
            return (y0 * Z + z0 + b_off) + x0 * (Y * Z)

        def stage1(n, b):
            coords_v, idx_v, vals_v, sem = coords_b[b], idx_b[b], vals_b[b], sem_b[b]
            p0 = base0 + n * K
            plane = p0 // NL
            s = p0 - plane * NL
            cbase = plane * (3 * NL) + s
            pltpu.sync_copy(coords.at[pl.ds(cbase, K)], coords_v.at[pl.ds(0, K)])
            pltpu.sync_copy(coords.at[pl.ds(cbase + NL, K)],
                            coords_v.at[pl.ds(K, K)])
            pltpu.sync_copy(coords.at[pl.ds(cbase + 2 * NL, K)],
                            coords_v.at[pl.ds(2 * K, K)])

            @pl.loop(0, K // _L)
            def pass1(jj):
                i0 = jj * _L
                x, y, z = load_xyz(coords_v, i0)
                m = mega_ids(x, y, z)
                pos = iota + i0
                plsc.store_scatter(idx_v, [pos >> 7, pos & 127], m >> 2)

            for g in range(GD):
                pltpu.async_copy(
                    table.at[idx_v.at[g]],
                    vals_v.at[pl.ds(g * 128, 128)],
                    sem,
                )

        def stage2(n, b):
            coords_v, idx_v, vals_v, sem = coords_b[b], idx_b[b], vals_b[b], sem_b[b]
            out_v = out_b[b]
            p0 = base0 + n * K
            for g in range(GD):
                pltpu.make_async_copy(
                    table.at[idx_v.at[g]],
                    vals_v.at[pl.ds(g * 128, 128)],
                    sem,
                ).wait()

            @pl.loop(0, K // _L)
            def pass2(jj):
                i0 = jj * _L
                x, y, z = load_xyz(coords_v, i0)
                m = mega_ids(x, y, z)
                colb = (m & 3) * MQ
                fx = x - x.astype(jnp.int32).astype(jnp.float32)
                fy = y - y.astype(jnp.int32).astype(jnp.float32)
                fz = z - z.astype(jnp.int32).astype(jnp.float32)
                gx = ones - fx
                gy = ones - fy
                gz = ones - fz
                wq = [gy * gz, gy * fz, fy * gz, fy * fz]
                wa = [gx, fx]
                rows0 = iota + i0
                acc = [zeros, zeros, zeros, zeros]
                for a in range(2):
                    cba = colb + a * (4 * C)
                    for c in range(C):
                        t = zeros
                        for q in range(4):
                            v = plsc.load_gather(
                                vals_v, [rows0, cba + (q * C + c)])
                            t = t + wq[q] * v
                        acc[c] = acc[c] + wa[a] * t
                line = i0 // 96
                within = i0 - line * 96
                for c in range(C):
                    out_v[pl.ds(line * (96 * C) + c * 96 + within, _L)] = acc[c]

            pltpu.sync_copy(out_v, out.at[pl.ds(4 * p0, 4 * K)])

        stage1(jnp.int32(0), 0)
        stage1(jnp.int32(1), 1)

        @pl.loop(0, NCHUNK // 2 - 1)
        def chunk_pair(m):
            n0 = 2 * m
            stage2(n0, 0)
            stage1(n0 + 2, 0)
            stage2(n0 + 1, 1)
            stage1(n0 + 3, 1)

        stage2(jnp.int32(NCHUNK - 2), 0)
        stage2(jnp.int32(NCHUNK - 1), 1)

    return pl.kernel(
        body,
        out_type=jax.ShapeDtypeStruct((P * C,), jnp.float32),
        mesh=mesh,
        scratch_types=[
            pltpu.VMEM((3 * K,), jnp.float32),
            pltpu.VMEM((3 * K,), jnp.float32),
            pltpu.VMEM((GD, 128), jnp.int32),
            pltpu.VMEM((GD, 128), jnp.int32),
            pltpu.VMEM((K, 128), jnp.float32),
            pltpu.VMEM((K, 128), jnp.float32),
            pltpu.VMEM((4 * K,), jnp.float32),
            pltpu.VMEM((4 * K,), jnp.float32),
            pltpu.SemaphoreType.DMA,
            pltpu.SemaphoreType.DMA,
        ],
        compiler_params=pltpu.CompilerParams(needs_layout_passes=False),
    )


def kernel(inputs, sample_coords):
    B, X, Y, Z, C = inputs.shape
    d0, d1, d2 = sample_coords.shape[1:4]
    P = B * d0 * d1 * d2
    NL = d1 * d2
    vol2d = inputs.transpose(0, 1, 2, 4, 3).reshape(B * X * Y * C, Z)
    table = _make_reformat(B, X, Y, Z, C)(vol2d)
    coords = sample_coords.transpose(0, 1, 4, 2, 3).reshape(P * 3)
    out = _make_kernel(B, X, Y, Z, C, P, NL)(table, coords)
    return out.reshape(B, d0, d1, C, d2).transpose(0, 1, 2, 4, 3)


# speedup vs baseline: 1.2743x; 1.2743x over previous
import jax
import jax.numpy as jnp
from jax import lax
from jax.experimental import pallas as pl
from jax.experimental.pallas import tpu as pltpu
from jax.experimental.pallas import tpu_sc as plsc

_NC = 2
_NS = 16
_NW = _NC * _NS
_L = 16


def _make_reformat(B, X, Y, Z, C):
    NLINE = B * X * Y
    LPW = NLINE // _NW
    SHEET = Y
    NSHEET = LPW // SHEET
    mesh = plsc.VectorSubcoreMesh(core_axis_name="c", subcore_axis_name="s")

    def body(vol2d, table, sheet_v, out_v, sem):
        cid = lax.axis_index("c")
        sid = lax.axis_index("s")
        wid = cid * _NS + sid
        line0 = wid * LPW
        j = lax.iota(jnp.int32, _L)
        q = j >> 2
        ch = j & 3
        dy = q >> 1
        c_clamp = q & 1
        c_hi = c_clamp + ch * Z
        c_lo = c_hi + dy * (C * Z)

        @pl.loop(0, NSHEET)
        def sheet_loop(s):
            sheet_line0 = line0 + s * SHEET
            pltpu.sync_copy(
                vol2d.at[pl.ds(sheet_line0 * C, SHEET * C)], sheet_v)

            @pl.loop(0, SHEET)
            def line_loop(y):
                cy = jnp.where(y < SHEET - 1, c_lo, c_hi) + y * (C * Z)
                cyz = cy - c_clamp

                @pl.loop(0, Z, unroll=8)
                def z_loop(z):
                    idx = jnp.where(z < Z - 1, cy, cyz) + z
                    v = plsc.load_gather(sheet_v, [idx >> 7, idx & (Z - 1)])
                    out_v[z >> 3, pl.ds((z & 7) * 16, 16)] = v

                pltpu.sync_copy(
                    out_v, table.at[pl.ds((sheet_line0 + y) * (Z // 8), Z // 8)])

    return pl.kernel(
        body,
        out_type=jax.ShapeDtypeStruct((B * X * Y * Z // 8, 8 * 4 * C), jnp.float32),
        mesh=mesh,
        scratch_types=[
            pltpu.VMEM((SHEET * C, Z), jnp.float32),
            pltpu.VMEM((Z // 8, 8 * 4 * C), jnp.float32),
            pltpu.SemaphoreType.DMA,
        ],
        compiler_params=pltpu.CompilerParams(needs_layout_passes=False, disable_bounds_checks=True),
    )


def _make_kernel(B, X, Y, Z, C, P, NL):
    PPW = P // _NW
    K = 192
    NCHUNK = PPW // K
    NIDX = 2 * K
    GD = NIDX // 128
    QC = 4 * C

    mesh = plsc.VectorSubcoreMesh(core_axis_name="c", subcore_axis_name="s")

    def body(table, coords, out,
             coords_v0, coords_v1, idx_v0, idx_v1, vals_v0, vals_v1,
             out_v0, out_v1, sem0, sem1):
        coords_b = [coords_v0, coords_v1]
        idx_b = [idx_v0, idx_v1]
        vals_b = [vals_v0, vals_v1]
        out_b = [out_v0, out_v1]
        sem_b = [sem0, sem1]
        cid = lax.axis_index("c")
        sid = lax.axis_index("s")
        wid = cid * _NS + sid
        batch = (wid * PPW) // (P // B)
        b_off = batch * (X * Y * Z)
        base0 = wid * PPW
        iota = lax.iota(jnp.int32, _L)
        zeros = jnp.zeros((_L,), jnp.float32)
        ones = jnp.ones((_L,), jnp.float32)

        def load_xyz(coords_v, i0):
            x = coords_v[pl.ds(i0, _L)]
            y = coords_v[pl.ds(K + i0, _L)]
            z = coords_v[pl.ds(2 * K + i0, _L)]
            return x, y, z

        def quad_ids(x, y, z):
            xi = x.astype(jnp.int32)
            yi = y.astype(jnp.int32)
            zi = z.astype(jnp.int32)
            x0 = jnp.clip(xi, 0, X - 1)
            x1 = jnp.clip(xi + 1, 0, X - 1)
            y0 = jnp.clip(yi, 0, Y - 1)
            z0 = jnp.clip(zi, 0, Z - 1)
            qbase = y0 * Z + z0 + b_off
            q0 = qbase + x0 * (Y * Z)
            q1 = qbase + x1 * (Y * Z)
            return q0, q1

        def stage1(n, b):
            coords_v, idx_v, vals_v, sem = coords_b[b], idx_b[b], vals_b[b], sem_b[b]
            p0 = base0 + n * K
            plane = p0 // NL
            s = p0 - plane * NL
            cbase = plane * (3 * NL) + s
            pltpu.sync_copy(coords.at[pl.ds(cbase, K)], coords_v.at[pl.ds(0, K)])
            pltpu.sync_copy(coords.at[pl.ds(cbase + NL, K)],
                            coords_v.at[pl.ds(K, K)])
            pltpu.sync_copy(coords.at[pl.ds(cbase + 2 * NL, K)],
                            coords_v.at[pl.ds(2 * K, K)])

            @pl.loop(0, K // _L)
            def pass1(jj):
                i0 = jj * _L
                x, y, z = load_xyz(coords_v, i0)
                q0, q1 = quad_ids(x, y, z)
                pos0 = iota + i0
                pos1 = pos0 + K
                plsc.store_scatter(idx_v, [pos0 >> 7, pos0 & 127], q0 >> 3)
                plsc.store_scatter(idx_v, [pos1 >> 7, pos1 & 127], q1 >> 3)

            for g in range(GD):
                pltpu.async_copy(
                    table.at[idx_v.at[g]],
                    vals_v.at[pl.ds(g * 128, 128)],
                    sem,
                )

        def stage2(n, b):
            coords_v, idx_v, vals_v, sem = coords_b[b], idx_b[b], vals_b[b], sem_b[b]
            out_v = out_b[b]
            p0 = base0 + n * K
            for g in range(GD):
                pltpu.make_async_copy(
                    table.at[idx_v.at[g]],
                    vals_v.at[pl.ds(g * 128, 128)],
                    sem,
                ).wait()

            @pl.loop(0, K // _L)
            def pass2(jj):
                i0 = jj * _L
                x, y, z = load_xyz(coords_v, i0)
                q0, q1 = quad_ids(x, y, z)
                colb = [(q0 & 7) * QC, (q1 & 7) * QC]
                fx = x - x.astype(jnp.int32).astype(jnp.float32)
                fy = y - y.astype(jnp.int32).astype(jnp.float32)
                fz = z - z.astype(jnp.int32).astype(jnp.float32)
                gx = ones - fx
                gy = ones - fy
                gz = ones - fz
                wq = [gy * gz, gy * fz, fy * gz, fy * fz]
                wa = [gx, fx]
                rows0 = iota + i0
                acc = [zeros, zeros, zeros, zeros]
                for a in range(2):
                    r = rows0 + a * K
                    for c in range(C):
                        t = zeros
                        for q in range(4):
                            v = plsc.load_gather(
                                vals_v, [r, colb[a] + (q * C + c)])
                            t = t + wq[q] * v
                        acc[c] = acc[c] + wa[a] * t
                line = i0 // 96
                within = i0 - line * 96
                for c in range(C):
                    out_v[pl.ds(line * (96 * C) + c * 96 + within, _L)] = acc[c]

            pltpu.sync_copy(out_v, out.at[pl.ds(4 * p0, 4 * K)])

        stage1(jnp.int32(0), 0)
        stage1(jnp.int32(1), 1)

        @pl.loop(0, NCHUNK // 2 - 1)
        def chunk_pair(m):
            n0 = 2 * m
            stage2(n0, 0)
            stage1(n0 + 2, 0)
            stage2(n0 + 1, 1)
            stage1(n0 + 3, 1)

        stage2(jnp.int32(NCHUNK - 2), 0)
        stage2(jnp.int32(NCHUNK - 1), 1)

    return pl.kernel(
        body,
        out_type=jax.ShapeDtypeStruct((P * C,), jnp.float32),
        mesh=mesh,
        scratch_types=[
            pltpu.VMEM((3 * K,), jnp.float32),
            pltpu.VMEM((3 * K,), jnp.float32),
            pltpu.VMEM((GD, 128), jnp.int32),
            pltpu.VMEM((GD, 128), jnp.int32),
            pltpu.VMEM((NIDX, 128), jnp.float32),
            pltpu.VMEM((NIDX, 128), jnp.float32),
            pltpu.VMEM((4 * K,), jnp.float32),
            pltpu.VMEM((4 * K,), jnp.float32),
            pltpu.SemaphoreType.DMA,
            pltpu.SemaphoreType.DMA,
        ],
        compiler_params=pltpu.CompilerParams(needs_layout_passes=False, disable_bounds_checks=True),
    )


def kernel(inputs, sample_coords):
    B, X, Y, Z, C = inputs.shape
    d0, d1, d2 = sample_coords.shape[1:4]
    P = B * d0 * d1 * d2
    NL = d1 * d2
    vol2d = inputs.transpose(0, 1, 2, 4, 3).reshape(B * X * Y * C, Z)
    table = _make_reformat(B, X, Y, Z, C)(vol2d)
    coords = sample_coords.transpose(0, 1, 4, 2, 3).reshape(P * 3)
    out = _make_kernel(B, X, Y, Z, C, P, NL)(table, coords)
    return out.reshape(B, d0, d1, C, d2).transpose(0, 1, 2, 4, 3)


# speedup vs baseline: 1.3608x; 1.0680x over previous
import jax
import jax.numpy as jnp
from jax import lax
from jax.experimental import pallas as pl
from jax.experimental.pallas import tpu as pltpu
from jax.experimental.pallas import tpu_sc as plsc

_NC = 2
_NS = 16
_NW = _NC * _NS
_L = 16

_PARAMS = pltpu.CompilerParams(
    needs_layout_passes=False, disable_bounds_checks=True)


def _make_reformat(B, X, Y, Z, C):
    NLINE = B * X * Y
    LPW = NLINE // _NW
    SHEET = Y
    NSHEET = LPW // SHEET
    LB = 4
    RPL = Z // 8
    mesh = plsc.VectorSubcoreMesh(core_axis_name="c", subcore_axis_name="s")

    def body(vol1d, table, sheet_v, ob0, ob1, sem0, sem1):
        obufs = [ob0, ob1]
        sems = [sem0, sem1]
        cid = lax.axis_index("c")
        sid = lax.axis_index("s")
        wid = cid * _NS + sid
        line0 = wid * LPW
        j = lax.iota(jnp.int32, _L)
        q = j >> 2
        ch = j & 3
        dy = q >> 1
        c_clamp = q & 1
        c_hi = c_clamp + ch * Z
        c_lo = c_hi + dy * (C * Z)

        @pl.loop(0, NSHEET)
        def sheet_loop(s):
            sheet_line0 = line0 + s * SHEET
            pltpu.sync_copy(
                vol1d.at[pl.ds(sheet_line0 * (C * Z), SHEET * C * Z)], sheet_v)

            @pl.loop(0, SHEET // (2 * LB))
            def line_blk(t):
                for bi in range(2):
                    ob, sem = obufs[bi], sems[bi]
                    yy0 = (2 * t + bi) * LB

                    @pl.when(jnp.logical_or(t > 0, s > 0))
                    def _():
                        pltpu.make_async_copy(
                            ob, table.at[pl.ds(0, LB * RPL)], sem).wait()

                    for li in range(LB):
                        yy = yy0 + li
                        cy = jnp.where(yy < SHEET - 1, c_lo, c_hi) + yy * (C * Z)
                        cyz = cy - c_clamp

                        @pl.loop(0, Z, unroll=8)
                        def z_loop(z):
                            idx = jnp.where(z < Z - 1, cy, cyz) + z
                            v = plsc.load_gather(sheet_v, [idx])
                            ob[li * RPL + (z >> 3), pl.ds((z & 7) * 16, 16)] = v

                    pltpu.async_copy(
                        ob,
                        table.at[pl.ds((sheet_line0 + yy0) * RPL, LB * RPL)],
                        sem)

        for bi in range(2):
            pltpu.make_async_copy(
                obufs[bi], table.at[pl.ds(0, LB * RPL)], sems[bi]).wait()

    return pl.kernel(
        body,
        out_type=jax.ShapeDtypeStruct((B * X * Y * Z // 8, 8 * 4 * C), jnp.float32),
        mesh=mesh,
        scratch_types=[
            pltpu.VMEM((SHEET * C * Z,), jnp.float32),
            pltpu.VMEM((LB * RPL, 128), jnp.float32),
            pltpu.VMEM((LB * RPL, 128), jnp.float32),
            pltpu.SemaphoreType.DMA,
            pltpu.SemaphoreType.DMA,
        ],
        compiler_params=_PARAMS,
    )


def _make_kernel(B, X, Y, Z, C, P, NL):
    PPW = P // _NW
    K = 192
    NCHUNK = PPW // K
    NIDX = 2 * K
    GD = NIDX // 128
    QC = 4 * C

    mesh = plsc.VectorSubcoreMesh(core_axis_name="c", subcore_axis_name="s")

    def body(table, coords, out,
             coords_v0, coords_v1, idx_v0, idx_v1, vals_v0, vals_v1,
             out_v0, out_v1, sem0, sem1):
        coords_b = [coords_v0, coords_v1]
        idx_b = [idx_v0, idx_v1]
        vals_b = [vals_v0, vals_v1]
        out_b = [out_v0, out_v1]
        sem_b = [sem0, sem1]
        cid = lax.axis_index("c")
        sid = lax.axis_index("s")
        wid = cid * _NS + sid
        batch = (wid * PPW) // (P // B)
        b_off = batch * (X * Y * Z)
        base0 = wid * PPW
        iota = lax.iota(jnp.int32, _L)
        zeros = jnp.zeros((_L,), jnp.float32)
        ones = jnp.ones((_L,), jnp.float32)

        def load_xyz(coords_v, i0):
            x = coords_v[pl.ds(i0, _L)]
            y = coords_v[pl.ds(K + i0, _L)]
            z = coords_v[pl.ds(2 * K + i0, _L)]
            return x, y, z

        def quad_ids(x, y, z):
            xi = x.astype(jnp.int32)
            yi = y.astype(jnp.int32)
            zi = z.astype(jnp.int32)
            x0 = jnp.clip(xi, 0, X - 1)
            x1 = jnp.clip(xi + 1, 0, X - 1)
            y0 = jnp.clip(yi, 0, Y - 1)
            z0 = jnp.clip(zi, 0, Z - 1)
            qbase = y0 * Z + z0 + b_off
            q0 = qbase + x0 * (Y * Z)
            q1 = qbase + x1 * (Y * Z)
            return q0, q1

        def stage1(n, b):
            coords_v, idx_v, sem = coords_b[b], idx_b[b], sem_b[b]
            vals_v = vals_b[b]
            p0 = base0 + n * K
            plane = p0 // NL
            s = p0 - plane * NL
            cbase = plane * (3 * NL) + s
            pltpu.sync_copy(coords.at[pl.ds(cbase, K)], coords_v.at[pl.ds(0, K)])
            pltpu.sync_copy(coords.at[pl.ds(cbase + NL, K)],
                            coords_v.at[pl.ds(K, K)])
            pltpu.sync_copy(coords.at[pl.ds(cbase + 2 * NL, K)],
                            coords_v.at[pl.ds(2 * K, K)])

            @pl.loop(0, K // _L)
            def pass1(jj):
                i0 = jj * _L
                x, y, z = load_xyz(coords_v, i0)
                q0, q1 = quad_ids(x, y, z)
                pos0 = iota + i0
                pos1 = pos0 + K
                plsc.store_scatter(idx_v, [pos0 >> 7, pos0 & 127], q0 >> 3)
                plsc.store_scatter(idx_v, [pos1 >> 7, pos1 & 127], q1 >> 3)

            for g in range(GD):
                pltpu.async_copy(
                    table.at[idx_v.at[g]],
                    vals_v.at[pl.ds(g * 128, 128)],
                    sem,
                )

        def stage2(n, b):
            coords_v, idx_v, sem = coords_b[b], idx_b[b], sem_b[b]
            vals_v = vals_b[b]
            out_v = out_b[b]
            p0 = base0 + n * K
            for g in range(GD):
                pltpu.make_async_copy(
                    table.at[idx_v.at[g]],
                    vals_v.at[pl.ds(g * 128, 128)],
                    sem,
                ).wait()

            @pl.loop(0, K // _L)
            def pass2(jj):
                i0 = jj * _L
                x, y, z = load_xyz(coords_v, i0)
                q0, q1 = quad_ids(x, y, z)
                fx = x - x.astype(jnp.int32).astype(jnp.float32)
                fy = y - y.astype(jnp.int32).astype(jnp.float32)
                fz = z - z.astype(jnp.int32).astype(jnp.float32)
                gx = ones - fx
                gy = ones - fy
                gz = ones - fz
                wq = [gy * gz, gy * fz, fy * gz, fy * fz]
                wa = [gx, fx]
                rows_a = [iota + i0, iota + (i0 + K)]
                colb_a = [(q0 & 7) * QC, (q1 & 7) * QC]
                acc = [zeros, zeros, zeros, zeros]
                for a in range(2):
                    for c in range(C):
                        t = zeros
                        for q in range(4):
                            v = plsc.load_gather(
                                vals_v, [rows_a[a], colb_a[a] + (q * C + c)])
                            t = t + wq[q] * v
                        acc[c] = acc[c] + wa[a] * t
                line = i0 // 96
                within = i0 - line * 96
                for c in range(C):
                    out_v[pl.ds(line * (96 * C) + c * 96 + within, _L)] = acc[c]

            pltpu.sync_copy(out_v, out.at[pl.ds(4 * p0, 4 * K)])

        stage1(jnp.int32(0), 0)
        stage1(jnp.int32(1), 1)

        @pl.loop(0, NCHUNK // 2 - 1)
        def chunk_pair(m):
            n0 = 2 * m
            stage2(n0, 0)
            stage1(n0 + 2, 0)
            stage2(n0 + 1, 1)
            stage1(n0 + 3, 1)

        stage2(jnp.int32(NCHUNK - 2), 0)
        stage2(jnp.int32(NCHUNK - 1), 1)

    return pl.kernel(
        body,
        out_type=jax.ShapeDtypeStruct((P * C,), jnp.float32),
        mesh=mesh,
        scratch_types=[
            pltpu.VMEM((3 * K,), jnp.float32),
            pltpu.VMEM((3 * K,), jnp.float32),
            pltpu.VMEM((GD, 128), jnp.int32),
            pltpu.VMEM((GD, 128), jnp.int32),
            pltpu.VMEM((NIDX, 128), jnp.float32),
            pltpu.VMEM((NIDX, 128), jnp.float32),
            pltpu.VMEM((4 * K,), jnp.float32),
            pltpu.VMEM((4 * K,), jnp.float32),
            pltpu.SemaphoreType.DMA,
            pltpu.SemaphoreType.DMA,
        ],
        compiler_params=_PARAMS,
    )


def kernel(inputs, sample_coords):
    B, X, Y, Z, C = inputs.shape
    d0, d1, d2 = sample_coords.shape[1:4]
    P = B * d0 * d1 * d2
    NL = d1 * d2
    vol1d = inputs.transpose(0, 1, 2, 4, 3).reshape(B * X * Y * C * Z)
    table = _make_reformat(B, X, Y, Z, C)(vol1d)
    coords = sample_coords.transpose(0, 1, 4, 2, 3).reshape(P * 3)
    out = _make_kernel(B, X, Y, Z, C, P, NL)(table, coords)
    return out.reshape(B, d0, d1, C, d2).transpose(0, 1, 2, 4, 3)


# speedup vs baseline: 1.5626x; 1.1483x over previous
import jax
import jax.numpy as jnp
from jax import lax
from jax.experimental import pallas as pl
from jax.experimental.pallas import tpu as pltpu
from jax.experimental.pallas import tpu_sc as plsc

_NC = 2
_NS = 16
_NW = _NC * _NS
_L = 16

_PARAMS = pltpu.CompilerParams(
    needs_layout_passes=False, disable_bounds_checks=True)


def _make_reformat(B, X, Y, Z, C):
    NLINE = B * X * Y
    LPW = NLINE // _NW
    SHEET = Y
    NSHEET = LPW // SHEET
    LB = 4
    RPL = Z // 8
    mesh = plsc.VectorSubcoreMesh(core_axis_name="c", subcore_axis_name="s")

    def body(vol1d, table, sheet_v, ob0, ob1, sem0, sem1):
        obufs = [ob0, ob1]
        sems = [sem0, sem1]
        cid = lax.axis_index("c")
        sid = lax.axis_index("s")
        wid = cid * _NS + sid
        line0 = wid * LPW
        j = lax.iota(jnp.int32, _L)
        q = j >> 2
        ch = j & 3
        dy = q >> 1
        c_clamp = q & 1
        c_hi = c_clamp + ch * Z
        c_lo = c_hi + dy * (C * Z)

        @pl.loop(0, NSHEET)
        def sheet_loop(s):
            sheet_line0 = line0 + s * SHEET
            pltpu.sync_copy(
                vol1d.at[pl.ds(sheet_line0 * (C * Z), SHEET * C * Z)], sheet_v)

            @pl.loop(0, SHEET // (2 * LB))
            def line_blk(t):
                for bi in range(2):
                    ob, sem = obufs[bi], sems[bi]
                    yy0 = (2 * t + bi) * LB

                    @pl.when(jnp.logical_or(t > 0, s > 0))
                    def _():
                        pltpu.make_async_copy(
                            ob, table.at[pl.ds(0, LB * RPL)], sem).wait()

                    for li in range(LB):
                        yy = yy0 + li
                        cy = jnp.where(yy < SHEET - 1, c_lo, c_hi) + yy * (C * Z)
                        cyz = cy - c_clamp

                        @pl.loop(0, Z, unroll=8)
                        def z_loop(z):
                            idx = jnp.where(z < Z - 1, cy, cyz) + z
                            v = plsc.load_gather(sheet_v, [idx])
                            ob[li * RPL + (z >> 3), pl.ds((z & 7) * 16, 16)] = v

                    pltpu.async_copy(
                        ob,
                        table.at[pl.ds((sheet_line0 + yy0) * RPL, LB * RPL)],
                        sem)

        for bi in range(2):
            pltpu.make_async_copy(
                obufs[bi], table.at[pl.ds(0, LB * RPL)], sems[bi]).wait()

    return pl.kernel(
        body,
        out_type=jax.ShapeDtypeStruct((B * X * Y * Z // 8, 8 * 4 * C), jnp.float32),
        mesh=mesh,
        scratch_types=[
            pltpu.VMEM((SHEET * C * Z,), jnp.float32),
            pltpu.VMEM((LB * RPL, 128), jnp.float32),
            pltpu.VMEM((LB * RPL, 128), jnp.float32),
            pltpu.SemaphoreType.DMA,
            pltpu.SemaphoreType.DMA,
        ],
        compiler_params=_PARAMS,
    )


def _make_kernel(B, X, Y, Z, C, P, NL):
    PPW = P // _NW
    K = 192
    NCHUNK = PPW // K
    NIDX = 2 * K
    GD = NIDX // 128
    QC = 4 * C

    mesh = plsc.VectorSubcoreMesh(core_axis_name="c", subcore_axis_name="s")

    def body(table, coords, out,
             coords_v0, coords_v1, idx_v0, idx_v1, vals_v0, vals_v1,
             out_v0, out_v1, sem0, sem1, semc0, semc1, semo0, semo1):
        coords_b = [coords_v0, coords_v1]
        idx_b = [idx_v0, idx_v1]
        vals_b = [vals_v0, vals_v1]
        out_b = [out_v0, out_v1]
        sem_b = [sem0, sem1]
        semc_b = [semc0, semc1]
        semo_b = [semo0, semo1]
        cid = lax.axis_index("c")
        sid = lax.axis_index("s")
        wid = cid * _NS + sid
        batch = (wid * PPW) // (P // B)
        b_off = batch * (X * Y * Z)
        base0 = wid * PPW
        iota = lax.iota(jnp.int32, _L)
        zeros = jnp.zeros((_L,), jnp.float32)
        ones = jnp.ones((_L,), jnp.float32)

        def load_xyz(coords_v, i0):
            x = coords_v[pl.ds(i0, _L)]
            y = coords_v[pl.ds(K + i0, _L)]
            z = coords_v[pl.ds(2 * K + i0, _L)]
            return x, y, z

        def quad_ids(x, y, z):
            xi = x.astype(jnp.int32)
            yi = y.astype(jnp.int32)
            zi = z.astype(jnp.int32)
            x0 = jnp.clip(xi, 0, X - 1)
            x1 = jnp.clip(xi + 1, 0, X - 1)
            y0 = jnp.clip(yi, 0, Y - 1)
            z0 = jnp.clip(zi, 0, Z - 1)
            qbase = y0 * Z + z0 + b_off
            q0 = qbase + x0 * (Y * Z)
            q1 = qbase + x1 * (Y * Z)
            return q0, q1

        def coord_copies(n, b):
            coords_v = coords_b[b]
            p0 = base0 + n * K
            plane = p0 // NL
            s = p0 - plane * NL
            cbase = plane * (3 * NL) + s
            return [
                (coords.at[pl.ds(cbase, K)], coords_v.at[pl.ds(0, K)]),
                (coords.at[pl.ds(cbase + NL, K)], coords_v.at[pl.ds(K, K)]),
                (coords.at[pl.ds(cbase + 2 * NL, K)], coords_v.at[pl.ds(2 * K, K)]),
            ]

        def fire_coords(n, b):
            for src, dst in coord_copies(n, b):
                pltpu.async_copy(src, dst, semc_b[b])

        def wait_coords(n, b):
            for src, dst in coord_copies(n, b):
                pltpu.make_async_copy(src, dst, semc_b[b]).wait()

        def pass1_fire(n, b):
            coords_v, idx_v, sem = coords_b[b], idx_b[b], sem_b[b]
            vals_v = vals_b[b]
            wait_coords(n, b)

            @pl.loop(0, K // _L)
            def pass1(jj):
                i0 = jj * _L
                x, y, z = load_xyz(coords_v, i0)
                q0, q1 = quad_ids(x, y, z)
                pos0 = iota + i0
                pos1 = pos0 + K
                plsc.store_scatter(idx_v, [pos0 >> 7, pos0 & 127], q0 >> 3)
                plsc.store_scatter(idx_v, [pos1 >> 7, pos1 & 127], q1 >> 3)

            for g in range(GD):
                pltpu.async_copy(
                    table.at[idx_v.at[g]],
                    vals_v.at[pl.ds(g * 128, 128)],
                    sem,
                )

        def pass2_do(n, b):
            coords_v, idx_v, sem = coords_b[b], idx_b[b], sem_b[b]
            vals_v = vals_b[b]
            out_v = out_b[b]
            p0 = base0 + n * K
            for g in range(GD):
                pltpu.make_async_copy(
                    table.at[idx_v.at[g]],
                    vals_v.at[pl.ds(g * 128, 128)],
                    sem,
                ).wait()

            @pl.when(n >= 2)
            def _():
                pltpu.make_async_copy(
                    out_v, out.at[pl.ds(0, 4 * K)], semo_b[b]).wait()

            @pl.loop(0, K // _L)
            def pass2(jj):
                i0 = jj * _L
                x, y, z = load_xyz(coords_v, i0)
                q0, q1 = quad_ids(x, y, z)
                fx = x - x.astype(jnp.int32).astype(jnp.float32)
                fy = y - y.astype(jnp.int32).astype(jnp.float32)
                fz = z - z.astype(jnp.int32).astype(jnp.float32)
                gx = ones - fx
                gy = ones - fy
                gz = ones - fz
                wq = [gy * gz, gy * fz, fy * gz, fy * fz]
                wa = [gx, fx]
                rows_a = [iota + i0, iota + (i0 + K)]
                colb_a = [(q0 & 7) * QC, (q1 & 7) * QC]
                acc = [zeros, zeros, zeros, zeros]
                for a in range(2):
                    for c in range(C):
                        t = zeros
                        for q in range(4):
                            v = plsc.load_gather(
                                vals_v, [rows_a[a], colb_a[a] + (q * C + c)])
                            t = t + wq[q] * v
                        acc[c] = acc[c] + wa[a] * t
                line = i0 // 96
                within = i0 - line * 96
                for c in range(C):
                    out_v[pl.ds(line * (96 * C) + c * 96 + within, _L)] = acc[c]

            pltpu.async_copy(out_v, out.at[pl.ds(4 * p0, 4 * K)], semo_b[b])

        fire_coords(jnp.int32(0), 0)
        fire_coords(jnp.int32(1), 1)
        pass1_fire(jnp.int32(0), 0)
        fire_coords(jnp.int32(2), 0)
        pass1_fire(jnp.int32(1), 1)
        fire_coords(jnp.int32(3), 1)

        @pl.loop(0, NCHUNK // 2 - 1)
        def chunk_pair(m):
            n0 = 2 * m
            pass2_do(n0, 0)
            pass1_fire(n0 + 2, 0)

            @pl.when(n0 + 4 < NCHUNK)
            def _():
                fire_coords(n0 + 4, 0)

            pass2_do(n0 + 1, 1)
            pass1_fire(n0 + 3, 1)

            @pl.when(n0 + 5 < NCHUNK)
            def _():
                fire_coords(n0 + 5, 1)

        pass2_do(jnp.int32(NCHUNK - 2), 0)
        pass2_do(jnp.int32(NCHUNK - 1), 1)
        for b in range(2):
            pltpu.make_async_copy(
                out_b[b], out.at[pl.ds(0, 4 * K)], semo_b[b]).wait()

    return pl.kernel(
        body,
        out_type=jax.ShapeDtypeStruct((P * C,), jnp.float32),
        mesh=mesh,
        scratch_types=[
            pltpu.VMEM((3 * K,), jnp.float32),
            pltpu.VMEM((3 * K,), jnp.float32),
            pltpu.VMEM((GD, 128), jnp.int32),
            pltpu.VMEM((GD, 128), jnp.int32),
            pltpu.VMEM((NIDX, 128), jnp.float32),
            pltpu.VMEM((NIDX, 128), jnp.float32),
            pltpu.VMEM((4 * K,), jnp.float32),
            pltpu.VMEM((4 * K,), jnp.float32),
            pltpu.SemaphoreType.DMA,
            pltpu.SemaphoreType.DMA,
            pltpu.SemaphoreType.DMA,
            pltpu.SemaphoreType.DMA,
            pltpu.SemaphoreType.DMA,
            pltpu.SemaphoreType.DMA,
        ],
        compiler_params=_PARAMS,
    )


def kernel(inputs, sample_coords):
    B, X, Y, Z, C = inputs.shape
    d0, d1, d2 = sample_coords.shape[1:4]
    P = B * d0 * d1 * d2
    NL = d1 * d2
    vol1d = inputs.transpose(0, 1, 2, 4, 3).reshape(B * X * Y * C * Z)
    table = _make_reformat(B, X, Y, Z, C)(vol1d)
    coords = sample_coords.transpose(0, 1, 4, 2, 3).reshape(P * 3)
    out = _make_kernel(B, X, Y, Z, C, P, NL)(table, coords)
    return out.reshape(B, d0, d1, C, d2).transpose(0, 1, 2, 4, 3)


# speedup vs baseline: 1.5945x; 1.0204x over previous
import jax
import jax.numpy as jnp
from jax import lax
from jax.experimental import pallas as pl
from jax.experimental.pallas import tpu as pltpu
from jax.experimental.pallas import tpu_sc as plsc

_NC = 2
_NS = 16
_NW = _NC * _NS
_L = 16

_PARAMS = pltpu.CompilerParams(
    needs_layout_passes=False, disable_bounds_checks=True)


def _make_reformat(B, X, Y, Z, C):
    NLINE = B * X * Y
    LPW = NLINE // _NW
    SHEET = Y
    NSHEET = LPW // SHEET
    LB = 4
    RPL = Z // 8
    mesh = plsc.VectorSubcoreMesh(core_axis_name="c", subcore_axis_name="s")

    def body(vol1d, table, sheet_v, ob0, ob1, sem0, sem1):
        obufs = [ob0, ob1]
        sems = [sem0, sem1]
        cid = lax.axis_index("c")
        sid = lax.axis_index("s")
        wid = cid * _NS + sid
        line0 = wid * LPW
        j = lax.iota(jnp.int32, _L)
        q = j >> 2
        ch = j & 3
        dy = q >> 1
        c_clamp = q & 1
        c_hi = c_clamp + ch * Z
        c_lo = c_hi + dy * (C * Z)

        @pl.loop(0, NSHEET)
        def sheet_loop(s):
            sheet_line0 = line0 + s * SHEET
            pltpu.sync_copy(
                vol1d.at[pl.ds(sheet_line0 * (C * Z), SHEET * C * Z)], sheet_v)

            @pl.loop(0, SHEET // (2 * LB))
            def line_blk(t):
                for bi in range(2):
                    ob, sem = obufs[bi], sems[bi]
                    yy0 = (2 * t + bi) * LB

                    @pl.when(jnp.logical_or(t > 0, s > 0))
                    def _():
                        pltpu.make_async_copy(
                            ob, table.at[pl.ds(0, LB * RPL)], sem).wait()

                    for li in range(LB):
                        yy = yy0 + li
                        cy = jnp.where(yy < SHEET - 1, c_lo, c_hi) + yy * (C * Z)
                        cyz = cy - c_clamp

                        @pl.loop(0, RPL)
                        def zr_loop(zr):
                            zb = zr * 8
                            base = cy + zb
                            base7 = jnp.where(zr == RPL - 1, cyz, cy) + zb
                            orow = li * RPL + zr
                            for k in range(7):
                                ob[orow, pl.ds(16 * k, 16)] = plsc.load_gather(
                                    sheet_v, [base + k])
                            ob[orow, pl.ds(16 * 7, 16)] = plsc.load_gather(
                                sheet_v, [base7 + 7])

                    pltpu.async_copy(
                        ob,
                        table.at[pl.ds((sheet_line0 + yy0) * RPL, LB * RPL)],
                        sem)

        for bi in range(2):
            pltpu.make_async_copy(
                obufs[bi], table.at[pl.ds(0, LB * RPL)], sems[bi]).wait()

    return pl.kernel(
        body,
        out_type=jax.ShapeDtypeStruct((B * X * Y * Z // 8, 8 * 4 * C), jnp.float32),
        mesh=mesh,
        scratch_types=[
            pltpu.VMEM((SHEET * C * Z,), jnp.float32),
            pltpu.VMEM((LB * RPL, 128), jnp.float32),
            pltpu.VMEM((LB * RPL, 128), jnp.float32),
            pltpu.SemaphoreType.DMA,
            pltpu.SemaphoreType.DMA,
        ],
        compiler_params=_PARAMS,
    )


def _make_kernel(B, X, Y, Z, C, P, NL):
    PPW = P // _NW
    K = 192
    NCHUNK = PPW // K
    NIDX = 2 * K
    GD = NIDX // 128
    QC = 4 * C

    mesh = plsc.VectorSubcoreMesh(core_axis_name="c", subcore_axis_name="s")

    def body(table, coords, out,
             coords_v0, coords_v1, idx_v0, idx_v1, vals_v0, vals_v1,
             out_v0, out_v1, wcol0, wcol1, wfrac0, wfrac1,
             sem0, sem1, semc0, semc1, semo0, semo1):
        coords_b = [coords_v0, coords_v1]
        idx_b = [idx_v0, idx_v1]
        vals_b = [vals_v0, vals_v1]
        out_b = [out_v0, out_v1]
        sem_b = [sem0, sem1]
        semc_b = [semc0, semc1]
        semo_b = [semo0, semo1]
        wcol_b = [wcol0, wcol1]
        wfrac_b = [wfrac0, wfrac1]
        cid = lax.axis_index("c")
        sid = lax.axis_index("s")
        wid = cid * _NS + sid
        batch = (wid * PPW) // (P // B)
        b_off = batch * (X * Y * Z)
        base0 = wid * PPW
        iota = lax.iota(jnp.int32, _L)
        zeros = jnp.zeros((_L,), jnp.float32)
        ones = jnp.ones((_L,), jnp.float32)

        def load_xyz(coords_v, i0):
            x = coords_v[pl.ds(i0, _L)]
            y = coords_v[pl.ds(K + i0, _L)]
            z = coords_v[pl.ds(2 * K + i0, _L)]
            return x, y, z

        def quad_ids(x, y, z):
            xi = x.astype(jnp.int32)
            yi = y.astype(jnp.int32)
            zi = z.astype(jnp.int32)
            x0 = jnp.clip(xi, 0, X - 1)
            x1 = jnp.clip(xi + 1, 0, X - 1)
            y0 = jnp.clip(yi, 0, Y - 1)
            z0 = jnp.clip(zi, 0, Z - 1)
            qbase = y0 * Z + z0 + b_off
            q0 = qbase + x0 * (Y * Z)
            q1 = qbase + x1 * (Y * Z)
            return q0, q1

        def coord_copies(n, b):
            coords_v = coords_b[b]
            p0 = base0 + n * K
            plane = p0 // NL
            s = p0 - plane * NL
            cbase = plane * (3 * NL) + s
            return [
                (coords.at[pl.ds(cbase, K)], coords_v.at[pl.ds(0, K)]),
                (coords.at[pl.ds(cbase + NL, K)], coords_v.at[pl.ds(K, K)]),
                (coords.at[pl.ds(cbase + 2 * NL, K)], coords_v.at[pl.ds(2 * K, K)]),
            ]

        def fire_coords(n, b):
            for src, dst in coord_copies(n, b):
                pltpu.async_copy(src, dst, semc_b[b])

        def wait_coords(n, b):
            for src, dst in coord_copies(n, b):
                pltpu.make_async_copy(src, dst, semc_b[b]).wait()

        def pass1_fire(n, b):
            coords_v, idx_v, sem = coords_b[b], idx_b[b], sem_b[b]
            vals_v = vals_b[b]
            wcol_v, wfrac_v = wcol_b[b], wfrac_b[b]
            wait_coords(n, b)

            @pl.loop(0, K // _L)
            def pass1(jj):
                i0 = jj * _L
                x, y, z = load_xyz(coords_v, i0)
                q0, q1 = quad_ids(x, y, z)
                pos0 = iota + i0
                pos1 = pos0 + K
                plsc.store_scatter(idx_v, [pos0 >> 7, pos0 & 127], q0 >> 3)
                plsc.store_scatter(idx_v, [pos1 >> 7, pos1 & 127], q1 >> 3)
                wcol_v[pl.ds(i0, _L)] = (q0 & 7) * QC
                wcol_v[pl.ds(K + i0, _L)] = (q1 & 7) * QC
                wfrac_v[pl.ds(i0, _L)] = x - x.astype(jnp.int32).astype(jnp.float32)
                wfrac_v[pl.ds(K + i0, _L)] = y - y.astype(jnp.int32).astype(jnp.float32)
                wfrac_v[pl.ds(2 * K + i0, _L)] = z - z.astype(jnp.int32).astype(jnp.float32)

            for g in range(GD):
                pltpu.async_copy(
                    table.at[idx_v.at[g]],
                    vals_v.at[pl.ds(g * 128, 128)],
                    sem,
                )

        def pass2_do(n, b):
            idx_v, sem = idx_b[b], sem_b[b]
            vals_v = vals_b[b]
            out_v = out_b[b]
            wcol_v, wfrac_v = wcol_b[b], wfrac_b[b]
            p0 = base0 + n * K
            for g in range(GD):
                pltpu.make_async_copy(
                    table.at[idx_v.at[g]],
                    vals_v.at[pl.ds(g * 128, 128)],
                    sem,
                ).wait()

            @pl.when(n >= 2)
            def _():
                pltpu.make_async_copy(
                    out_v, out.at[pl.ds(0, 4 * K)], semo_b[b]).wait()

            @pl.loop(0, K // _L)
            def pass2(jj):
                i0 = jj * _L
                fx = wfrac_v[pl.ds(i0, _L)]
                fy = wfrac_v[pl.ds(K + i0, _L)]
                fz = wfrac_v[pl.ds(2 * K + i0, _L)]
                gx = ones - fx
                gy = ones - fy
                gz = ones - fz
                wq = [gy * gz, gy * fz, fy * gz, fy * fz]
                wa = [gx, fx]
                rows_a = [iota + i0, iota + (i0 + K)]
                colb_a = [wcol_v[pl.ds(i0, _L)], wcol_v[pl.ds(K + i0, _L)]]
                acc = [zeros, zeros, zeros, zeros]
                for a in range(2):
                    for c in range(C):
                        t = zeros
                        for q in range(4):
                            v = plsc.load_gather(
                                vals_v, [rows_a[a], colb_a[a] + (q * C + c)])
                            t = t + wq[q] * v
                        acc[c] = acc[c] + wa[a] * t
                line = i0 // 96
                within = i0 - line * 96
                for c in range(C):
                    out_v[pl.ds(line * (96 * C) + c * 96 + within, _L)] = acc[c]

            pltpu.async_copy(out_v, out.at[pl.ds(4 * p0, 4 * K)], semo_b[b])

        fire_coords(jnp.int32(0), 0)
        fire_coords(jnp.int32(1), 1)
        pass1_fire(jnp.int32(0), 0)
        fire_coords(jnp.int32(2), 0)
        pass1_fire(jnp.int32(1), 1)
        fire_coords(jnp.int32(3), 1)

        @pl.loop(0, NCHUNK // 2 - 1)
        def chunk_pair(m):
            n0 = 2 * m
            pass2_do(n0, 0)
            pass1_fire(n0 + 2, 0)

            @pl.when(n0 + 4 < NCHUNK)
            def _():
                fire_coords(n0 + 4, 0)

            pass2_do(n0 + 1, 1)
            pass1_fire(n0 + 3, 1)

            @pl.when(n0 + 5 < NCHUNK)
            def _():
                fire_coords(n0 + 5, 1)

        pass2_do(jnp.int32(NCHUNK - 2), 0)
        pass2_do(jnp.int32(NCHUNK - 1), 1)
        for b in range(2):
            pltpu.make_async_copy(
                out_b[b], out.at[pl.ds(0, 4 * K)], semo_b[b]).wait()

    return pl.kernel(
        body,
        out_type=jax.ShapeDtypeStruct((P * C,), jnp.float32),
        mesh=mesh,
        scratch_types=[
            pltpu.VMEM((3 * K,), jnp.float32),
            pltpu.VMEM((3 * K,), jnp.float32),
            pltpu.VMEM((GD, 128), jnp.int32),
            pltpu.VMEM((GD, 128), jnp.int32),
            pltpu.VMEM((NIDX, 128), jnp.float32),
            pltpu.VMEM((NIDX, 128), jnp.float32),
            pltpu.VMEM((4 * K,), jnp.float32),
            pltpu.VMEM((4 * K,), jnp.float32),
            pltpu.VMEM((2 * K,), jnp.int32),
            pltpu.VMEM((2 * K,), jnp.int32),
            pltpu.VMEM((3 * K,), jnp.float32),
            pltpu.VMEM((3 * K,), jnp.float32),
            pltpu.SemaphoreType.DMA,
            pltpu.SemaphoreType.DMA,
            pltpu.SemaphoreType.DMA,
            pltpu.SemaphoreType.DMA,
            pltpu.SemaphoreType.DMA,
            pltpu.SemaphoreType.DMA,
        ],
        compiler_params=_PARAMS,
    )


def kernel(inputs, sample_coords):
    B, X, Y, Z, C = inputs.shape
    d0, d1, d2 = sample_coords.shape[1:4]
    P = B * d0 * d1 * d2
    NL = d1 * d2
    vol1d = inputs.transpose(0, 1, 2, 4, 3).reshape(B * X * Y * C * Z)
    table = _make_reformat(B, X, Y, Z, C)(vol1d)
    coords = sample_coords.transpose(0, 1, 4, 2, 3).reshape(P * 3)
    out = _make_kernel(B, X, Y, Z, C, P, NL)(table, coords)
    return out.reshape(B, d0, d1, C, d2).transpose(0, 1, 2, 4, 3)
